# Initial kernel scaffold; baseline (speedup 1.0000x reference)
import jax
import jax.numpy as jnp
from jax.experimental import pallas as pl


def kernel(q, k, v, node_features, edge_index, Wq, bq, Wk, bk, Wv, bv, Wo, bo, ln_g, ln_b, fc_W, fc_b, g1_W, g1_b, g2_W, g2_b):
    raise NotImplementedError("write your pallas kernel here")


# speedup vs baseline: 15.6702x; 15.6702x over previous
import jax
import jax.numpy as jnp
from jax import lax
from jax.experimental import pallas as pl
from jax.experimental.pallas import tpu as pltpu
from jax.experimental.pallas import tpu_sc as plsc

D_MODEL = 512
N_C = 1024
N_KNOW = 512
BS = 16
NF_DIM = 128
IN_CH = 256
N_CLS = 5
N_EDGES = 262144
NUM_NODES = BS * N_C

NCORE = 2
NSUB = 16
LANES = 16
CH = 128
NCHUNKS = N_EDGES // CH
CPT_ALL = NCHUNKS // NSUB
CPT_HALF = NCHUNKS // (NSUB * NCORE)
NBUF = 3
HALF_NODES = NUM_NODES // NCORE
ACC_ROWS = HALF_NODES + CH


def _dot_t(a, b):
  return lax.dot_general(a, b, (((1,), (1,)), ((), ())),
                         preferred_element_type=jnp.float32)


def _scmesh():
  return plsc.VectorSubcoreMesh(core_axis_name="c", subcore_axis_name="s",
                                num_cores=NCORE, num_subcores=NSUB)


def _attn_body(q_ref, k_ref, v_ref, nf_ref, wq_ref, bq_ref, wk_ref, bk_ref,
               wv_ref, bv_ref, wo_ref, bo_ref, lng_ref, lnb_ref, fcw_ref,
               fcb_ref, x0_ref):
  q = q_ref[0]
  k = k_ref[0]
  v = v_ref[0]
  qh = _dot_t(q, wq_ref[...]) + bq_ref[...][None, :]
  kh = _dot_t(k, wk_ref[...]) + bk_ref[...][None, :]
  vh = _dot_t(v, wv_ref[...]) + bv_ref[...][None, :]
  scores = _dot_t(qh, kh) * (1.0 / (D_MODEL ** 0.5))
  m = jnp.max(scores, axis=-1, keepdims=True)
  e = jnp.exp(scores - m)
  attn = e / jnp.sum(e, axis=-1, keepdims=True)
  ctx = jnp.dot(attn, vh, preferred_element_type=jnp.float32)
  out = _dot_t(ctx, wo_ref[...]) + bo_ref[...][None, :]
  qq = q + out
  mu = jnp.mean(qq, axis=-1, keepdims=True)
  var = jnp.mean((qq - mu) ** 2, axis=-1, keepdims=True)
  xln = (qq - mu) / jnp.sqrt(var + 1e-5) * lng_ref[...][None, :] + \
      lnb_ref[...][None, :]
  xcat = jnp.concatenate([xln, nf_ref[...]], axis=1)
  x0_ref[...] = _dot_t(xcat, fcw_ref[...]) + fcb_ref[...][None, :]


def _attn_call(q, k, v, nf, wq, bq, wk, bk, wv, bv, wo, bo, lng, lnb, fcw,
               fcb):
  full = lambda shape: pl.BlockSpec(shape, lambda i: tuple(0 for _ in shape))
  return pl.pallas_call(
      _attn_body,
      grid=(BS,),
      in_specs=[
          pl.BlockSpec((1, N_C, D_MODEL), lambda i: (i, 0, 0)),
          pl.BlockSpec((1, N_KNOW, D_MODEL), lambda i: (i, 0, 0)),
          pl.BlockSpec((1, N_KNOW, D_MODEL), lambda i: (i, 0, 0)),
          full((N_C, NF_DIM)),
          full((D_MODEL, D_MODEL)), full((D_MODEL,)),
          full((D_MODEL, D_MODEL)), full((D_MODEL,)),
          full((D_MODEL, D_MODEL)), full((D_MODEL,)),
          full((D_MODEL, D_MODEL)), full((D_MODEL,)),
          full((D_MODEL,)), full((D_MODEL,)),
          full((IN_CH, D_MODEL + NF_DIM)), full((IN_CH,)),
      ],
      out_specs=pl.BlockSpec((N_C, IN_CH), lambda i: (i, 0)),
      out_shape=jax.ShapeDtypeStruct((NUM_NODES, IN_CH), jnp.float32),
  )(q, k, v, nf, wq, bq, wk, bk, wv, bv, wo, bo, lng, lnb, fcw, fcb)


GRP = 16
NGRP = CPT_ALL // GRP
ZPT = ACC_ROWS // NSUB
WPT = HALF_NODES // NSUB


def _make_prop_body(with_gather):
  def body(*args):
    if with_gather:
      (x_hbm, src_hbm, dst_hbm, out_hbm, srcv, dstloc, rowbuf, iotav, acc,
       sem_g, sem_s, sem_i) = args
    else:
      (dst_hbm, out_hbm, dstloc, rowbuf, iotav, acc,
       sem_s, sem_i) = args
    cid = lax.axis_index("c")
    sid = lax.axis_index("s")
    base = cid * HALF_NODES
    iota = lax.iota(jnp.int32, LANES)
    gbase = sid * CPT_ALL

    def load_group(g, bank, sem):
      ds = pl.ds(gbase + g * GRP, GRP)
      descs = [pltpu.async_copy(dst_hbm.at[ds], dstloc.at[bank], sem)]
      if with_gather:
        descs.append(pltpu.async_copy(src_hbm.at[ds], srcv.at[bank], sem))
      return descs

    def remap_bank(bank):
      def remap(r, carry):
        for vv in range(CH // LANES):
          d = dstloc[bank, r, pl.ds(vv * LANES, LANES)]
          local = d - base
          ok = (local >= 0) & (local < HALF_NODES)
          dummy = HALF_NODES + ((iota + r + vv) & (CH - 1))
          dstloc[bank, r, pl.ds(vv * LANES, LANES)] = jnp.where(
              ok, local, dummy)
        return carry
      lax.fori_loop(0, GRP, remap, 0)

    def ifill(z, carry):
      for vv in range(CH // LANES):
        sl = pl.ds(vv * LANES, LANES)
        lane = vv * LANES + iota
        iotav[z, sl] = sid * ZPT + z * CH + lane
        iotav[5 + z, sl] = sid * WPT + z * CH + lane
      return carry
    lax.fori_loop(0, 4, ifill, 0)
    for vv in range(CH // LANES):
      sl = pl.ds(vv * LANES, LANES)
      lane = vv * LANES + iota
      iotav[4, sl] = jnp.where(4 * CH + lane < ZPT,
                               sid * ZPT + 4 * CH + lane,
                               sid * ZPT + (lane & 7))

    def zfill(i, carry):
      r = i // 8
      c = (i % 8) * LANES
      rowbuf[0, r, pl.ds(c, LANES)] = jnp.zeros((LANES,), jnp.float32)
      return carry
    lax.fori_loop(0, CH * 8, zfill, 0)
    for z in range(5):
      pltpu.async_copy(rowbuf.at[0], acc.at[iotav.at[z]], sem_s.at[0]).wait()
    if not with_gather:
      def ofill(i, carry):
        r = i // 8
        c = (i % 8) * LANES
        rowbuf[0, r, pl.ds(c, LANES)] = jnp.ones((LANES,), jnp.float32)
        return carry
      lax.fori_loop(0, CH * 8, ofill, 0)
    plsc.subcore_barrier()

    for d in load_group(0, 0, sem_i.at[0]):
      d.wait()
    remap_bank(0)

    gd = [None] * NBUF
    sd = [None] * NBUF
    idx_pending = []
    for j in range(CPT_ALL + 2):
      if j < CPT_ALL:
        g, pos = j // GRP, j % GRP
        bank = g % 2
        if pos == 0 and j > 0:
          for d in idx_pending:
            d.wait()
          idx_pending = []
          remap_bank(bank)
        if pos == 3 and g + 1 < NGRP:
          idx_pending = load_group(g + 1, (g + 1) % 2, sem_i.at[(g + 1) % 2])
        if with_gather:
          b = j % NBUF
          if j >= NBUF:
            sd[b].wait()
          gd[b] = pltpu.async_copy(x_hbm.at[srcv.at[bank, pos]],
                                   rowbuf.at[b], sem_g.at[b])
        else:
          b = 0
          if j >= NBUF:
            sd[j % NBUF].wait()
      if j >= 2:
        jj = j - 2
        gg, ppos = jj // GRP, jj % GRP
        bb = (jj % NBUF) if with_gather else 0
        if with_gather:
          gd[bb].wait()
        sd[jj % NBUF] = pltpu.async_copy(rowbuf.at[bb],
                                         acc.at[dstloc.at[gg % 2, ppos]],
                                         sem_s.at[jj % NBUF], add=True)
    for jj in range(CPT_ALL - NBUF, CPT_ALL):
      sd[jj % NBUF].wait()
    plsc.subcore_barrier()

    for z in range(4):
      pltpu.async_copy(acc.at[iotav.at[5 + z]], rowbuf.at[0],
                       sem_s.at[0]).wait()
      pltpu.sync_copy(rowbuf.at[0],
                      out_hbm.at[pl.ds(base + sid * WPT + z * CH, CH)])
  return body


def _prop_call(x, src2d, dst2d):
  return pl.kernel(
      _make_prop_body(True),
      out_type=jax.ShapeDtypeStruct((NUM_NODES, CH), jnp.float32),
      mesh=_scmesh(),
      scratch_types=[
          pltpu.VMEM((2, GRP, CH), jnp.int32),
          pltpu.VMEM((2, GRP, CH), jnp.int32),
          pltpu.VMEM((NBUF, CH, CH), jnp.float32),
          pltpu.VMEM((9, CH), jnp.int32),
          pltpu.VMEM_SHARED((ACC_ROWS, CH), jnp.float32),
          pltpu.SemaphoreType.DMA((NBUF,)),
          pltpu.SemaphoreType.DMA((NBUF,)),
          pltpu.SemaphoreType.DMA((2,)),
      ],
  )(x, src2d, dst2d)


def _deg_call(dst2d):
  return pl.kernel(
      _make_prop_body(False),
      out_type=jax.ShapeDtypeStruct((NUM_NODES, CH), jnp.float32),
      mesh=_scmesh(),
      scratch_types=[
          pltpu.VMEM((2, GRP, CH), jnp.int32),
          pltpu.VMEM((1, CH, CH), jnp.float32),
          pltpu.VMEM((9, CH), jnp.int32),
          pltpu.VMEM_SHARED((ACC_ROWS, CH), jnp.float32),
          pltpu.SemaphoreType.DMA((NBUF,)),
          pltpu.SemaphoreType.DMA((2,)),
      ],
  )(dst2d)


def _b_body(x0_ref, degf_ref, w1_ref, th0, th1, dinv_ref):
  deg = degf_ref[:, 0] + 1.0
  dinv = lax.rsqrt(deg)
  t = _dot_t(x0_ref[...], w1_ref[...])
  ts = t * dinv[:, None]
  dinv_ref[...] = dinv[:, None]
  th0[...] = ts[:, :128]
  th1[...] = ts[:, 128:]


def _b_call(x0, degf, w1):
  rows = N_C
  full = lambda shape: pl.BlockSpec(shape, lambda i: tuple(0 for _ in shape))
  return pl.pallas_call(
      _b_body,
      grid=(NUM_NODES // rows,),
      in_specs=[
          pl.BlockSpec((rows, IN_CH), lambda i: (i, 0)),
          pl.BlockSpec((rows, CH), lambda i: (i, 0)),
          full((IN_CH, IN_CH)),
      ],
      out_specs=[pl.BlockSpec((rows, 128), lambda i: (i, 0))] * 2 +
                [pl.BlockSpec((rows, 1), lambda i: (i, 0))],
      out_shape=[jax.ShapeDtypeStruct((NUM_NODES, 128), jnp.float32)] * 2 +
                [jax.ShapeDtypeStruct((NUM_NODES, 1), jnp.float32)],
  )(x0, degf, w1)


def _c_body(s0, s1, t0, t1, dinv_ref, b1_ref, w2_ref, us_ref):
  dinv = dinv_ref[...]
  h = jnp.concatenate(
      [(s0[...] + t0[...]) * dinv, (s1[...] + t1[...]) * dinv], axis=1)
  h = jnp.maximum(h + b1_ref[...][None, :], 0.0)
  u = _dot_t(h, w2_ref[...])
  us_ref[...] = u * dinv


def _c_call(s_halves, t_halves, dinv, b1, w2p):
  rows = N_C
  full = lambda shape: pl.BlockSpec(shape, lambda i: tuple(0 for _ in shape))
  return pl.pallas_call(
      _c_body,
      grid=(NUM_NODES // rows,),
      in_specs=[pl.BlockSpec((rows, 128), lambda i: (i, 0))] * 4 +
               [pl.BlockSpec((rows, 1), lambda i: (i, 0)),
                full((IN_CH,)), full((CH, IN_CH))],
      out_specs=pl.BlockSpec((rows, CH), lambda i: (i, 0)),
      out_shape=jax.ShapeDtypeStruct((NUM_NODES, CH), jnp.float32),
  )(*s_halves, *t_halves, dinv, b1, w2p)


def _e_body(s2_ref, us_ref, dinv_ref, b2_ref, out_ref):
  res = (s2_ref[...] + us_ref[...]) * dinv_ref[...] + b2_ref[...][None, :]
  out_ref[...] = res[:, :N_CLS]


def _e_call(s2, us, dinv, b2p):
  rows = N_C
  full = lambda shape: pl.BlockSpec(shape, lambda i: tuple(0 for _ in shape))
  return pl.pallas_call(
      _e_body,
      grid=(NUM_NODES // rows,),
      in_specs=[
          pl.BlockSpec((rows, CH), lambda i: (i, 0)),
          pl.BlockSpec((rows, CH), lambda i: (i, 0)),
          pl.BlockSpec((rows, 1), lambda i: (i, 0)),
          full((CH,)),
      ],
      out_specs=pl.BlockSpec((rows, N_CLS), lambda i: (i, 0)),
      out_shape=jax.ShapeDtypeStruct((NUM_NODES, N_CLS), jnp.float32),
  )(s2, us, dinv, b2p)


def kernel(q, k, v, node_features, edge_index, Wq, bq, Wk, bk, Wv, bv, Wo, bo,
           ln_g, ln_b, fc_W, fc_b, g1_W, g1_b, g2_W, g2_b):
  src2d = edge_index[0].astype(jnp.int32).reshape(NCHUNKS, CH)
  dst2d = edge_index[1].astype(jnp.int32).reshape(NCHUNKS, CH)

  degf = _deg_call(dst2d)
  x0 = _attn_call(q, k, v, node_features, Wq, bq, Wk, bk, Wv, bv, Wo, bo,
                  ln_g, ln_b, fc_W, fc_b)

  th0, th1, dinv = _b_call(x0, degf, g1_W)
  s0 = _prop_call(th0, src2d, dst2d)
  s1 = _prop_call(th1, src2d, dst2d)

  w2p = jnp.pad(g2_W, ((0, CH - N_CLS), (0, 0)))
  b2p = jnp.pad(g2_b, (0, CH - N_CLS))
  usp = _c_call((s0, s1), (th0, th1), dinv, g1_b, w2p)
  s2 = _prop_call(usp, src2d, dst2d)
  out = _e_call(s2, usp, dinv, b2p)
  return out.reshape(BS, N_C, N_CLS)


# speedup vs baseline: 15.6707x; 1.0000x over previous
import jax
import jax.numpy as jnp
from jax import lax
from jax.experimental import pallas as pl
from jax.experimental.pallas import tpu as pltpu
from jax.experimental.pallas import tpu_sc as plsc

D_MODEL = 512
N_C = 1024
N_KNOW = 512
BS = 16
NF_DIM = 128
IN_CH = 256
N_CLS = 5
N_EDGES = 262144
NUM_NODES = BS * N_C

NCORE = 2
NSUB = 16
LANES = 16
CH = 128
NCHUNKS = N_EDGES // CH
CPT_ALL = NCHUNKS // NSUB
CPT_HALF = NCHUNKS // (NSUB * NCORE)
NBUF = 3
HALF_NODES = NUM_NODES // NCORE
ACC_ROWS = HALF_NODES + CH


def _dot_t(a, b):
  return lax.dot_general(a, b, (((1,), (1,)), ((), ())),
                         preferred_element_type=jnp.float32)


def _dot_tb(a, b):
  return lax.dot_general(a.astype(jnp.bfloat16), b.astype(jnp.bfloat16),
                         (((1,), (1,)), ((), ())),
                         preferred_element_type=jnp.float32)


def _scmesh():
  return plsc.VectorSubcoreMesh(core_axis_name="c", subcore_axis_name="s",
                                num_cores=NCORE, num_subcores=NSUB)


def _attn_body(q_ref, k_ref, v_ref, nf_ref, wq_ref, bq_ref, wk_ref, bk_ref,
               wv_ref, bv_ref, wo_ref, bo_ref, lng_ref, lnb_ref, fcw_ref,
               fcb_ref, x0_ref):
  q = q_ref[0]
  k = k_ref[0]
  v = v_ref[0]
  qh = _dot_tb(q, wq_ref[...]) + bq_ref[...][None, :]
  kh = _dot_tb(k, wk_ref[...]) + bk_ref[...][None, :]
  vh = _dot_tb(v, wv_ref[...]) + bv_ref[...][None, :]
  scores = _dot_tb(qh, kh) * (1.0 / (D_MODEL ** 0.5))
  m = jnp.max(scores, axis=-1, keepdims=True)
  e = jnp.exp(scores - m)
  attn = e / jnp.sum(e, axis=-1, keepdims=True)
  ctx = lax.dot_general(attn.astype(jnp.bfloat16), vh.astype(jnp.bfloat16),
                        (((1,), (0,)), ((), ())),
                        preferred_element_type=jnp.float32)
  out = _dot_tb(ctx, wo_ref[...]) + bo_ref[...][None, :]
  qq = q + out
  mu = jnp.mean(qq, axis=-1, keepdims=True)
  var = jnp.mean((qq - mu) ** 2, axis=-1, keepdims=True)
  xln = (qq - mu) / jnp.sqrt(var + 1e-5) * lng_ref[...][None, :] + \
      lnb_ref[...][None, :]
  xcat = jnp.concatenate([xln, nf_ref[...]], axis=1)
  x0_ref[...] = _dot_tb(xcat, fcw_ref[...]) + fcb_ref[...][None, :]


def _attn_call(q, k, v, nf, wq, bq, wk, bk, wv, bv, wo, bo, lng, lnb, fcw,
               fcb):
  full = lambda shape: pl.BlockSpec(shape, lambda i: tuple(0 for _ in shape))
  return pl.pallas_call(
      _attn_body,
      grid=(BS,),
      in_specs=[
          pl.BlockSpec((1, N_C, D_MODEL), lambda i: (i, 0, 0)),
          pl.BlockSpec((1, N_KNOW, D_MODEL), lambda i: (i, 0, 0)),
          pl.BlockSpec((1, N_KNOW, D_MODEL), lambda i: (i, 0, 0)),
          full((N_C, NF_DIM)),
          full((D_MODEL, D_MODEL)), full((D_MODEL,)),
          full((D_MODEL, D_MODEL)), full((D_MODEL,)),
          full((D_MODEL, D_MODEL)), full((D_MODEL,)),
          full((D_MODEL, D_MODEL)), full((D_MODEL,)),
          full((D_MODEL,)), full((D_MODEL,)),
          full((IN_CH, D_MODEL + NF_DIM)), full((IN_CH,)),
      ],
      out_specs=pl.BlockSpec((N_C, IN_CH), lambda i: (i, 0)),
      out_shape=jax.ShapeDtypeStruct((NUM_NODES, IN_CH), jnp.float32),
  )(q, k, v, nf, wq, bq, wk, bk, wv, bv, wo, bo, lng, lnb, fcw, fcb)


GRP = 16
NGRP = CPT_ALL // GRP
ZPT = ACC_ROWS // NSUB
WPT = HALF_NODES // NSUB


def _make_prop_body(with_gather):
  def body(*args):
    if with_gather:
      (x_hbm, src_hbm, dst_hbm, out_hbm, srcv, dstloc, rowbuf, iotav, acc,
       sem_g, sem_s, sem_i) = args
    else:
      (dst_hbm, out_hbm, dstloc, rowbuf, iotav, acc,
       sem_s, sem_i) = args
    cid = lax.axis_index("c")
    sid = lax.axis_index("s")
    base = cid * HALF_NODES
    iota = lax.iota(jnp.int32, LANES)
    gbase = sid * CPT_ALL

    def load_group(g, bank, sem):
      ds = pl.ds(gbase + g * GRP, GRP)
      descs = [pltpu.async_copy(dst_hbm.at[ds], dstloc.at[bank], sem)]
      if with_gather:
        descs.append(pltpu.async_copy(src_hbm.at[ds], srcv.at[bank], sem))
      return descs

    def remap_bank(bank):
      def remap(r, carry):
        for vv in range(CH // LANES):
          d = dstloc[bank, r, pl.ds(vv * LANES, LANES)]
          local = d - base
          ok = (local >= 0) & (local < HALF_NODES)
          dummy = HALF_NODES + ((iota + r + vv) & (CH - 1))
          dstloc[bank, r, pl.ds(vv * LANES, LANES)] = jnp.where(
              ok, local, dummy)
        return carry
      lax.fori_loop(0, GRP, remap, 0)

    def ifill(z, carry):
      for vv in range(CH // LANES):
        sl = pl.ds(vv * LANES, LANES)
        lane = vv * LANES + iota
        iotav[z, sl] = sid * ZPT + z * CH + lane
        iotav[5 + z, sl] = sid * WPT + z * CH + lane
      return carry
    lax.fori_loop(0, 4, ifill, 0)
    for vv in range(CH // LANES):
      sl = pl.ds(vv * LANES, LANES)
      lane = vv * LANES + iota
      iotav[4, sl] = jnp.where(4 * CH + lane < ZPT,
                               sid * ZPT + 4 * CH + lane,
                               sid * ZPT + (lane & 7))

    def zfill(i, carry):
      r = i // 8
      c = (i % 8) * LANES
      rowbuf[0, r, pl.ds(c, LANES)] = jnp.zeros((LANES,), jnp.float32)
      return carry
    lax.fori_loop(0, CH * 8, zfill, 0)
    for z in range(5):
      pltpu.async_copy(rowbuf.at[0], acc.at[iotav.at[z]], sem_s.at[0]).wait()
    if not with_gather:
      def ofill(i, carry):
        r = i // 8
        c = (i % 8) * LANES
        rowbuf[0, r, pl.ds(c, LANES)] = jnp.ones((LANES,), jnp.float32)
        return carry
      lax.fori_loop(0, CH * 8, ofill, 0)
    plsc.subcore_barrier()

    for d in load_group(0, 0, sem_i.at[0]):
      d.wait()
    remap_bank(0)

    gd = [None] * NBUF
    sd = [None] * NBUF
    idx_pending = []
    for j in range(CPT_ALL + 2):
      if j < CPT_ALL:
        g, pos = j // GRP, j % GRP
        bank = g % 2
        if pos == 0 and j > 0:
          for d in idx_pending:
            d.wait()
          idx_pending = []
          remap_bank(bank)
        if pos == 3 and g + 1 < NGRP:
          idx_pending = load_group(g + 1, (g + 1) % 2, sem_i.at[(g + 1) % 2])
        if with_gather:
          b = j % NBUF
          if j >= NBUF:
            sd[b].wait()
          gd[b] = pltpu.async_copy(x_hbm.at[srcv.at[bank, pos]],
                                   rowbuf.at[b], sem_g.at[b])
        else:
          b = 0
          if j >= NBUF:
            sd[j % NBUF].wait()
      if j >= 2:
        jj = j - 2
        gg, ppos = jj // GRP, jj % GRP
        bb = (jj % NBUF) if with_gather else 0
        if with_gather:
          gd[bb].wait()
        sd[jj % NBUF] = pltpu.async_copy(rowbuf.at[bb],
                                         acc.at[dstloc.at[gg % 2, ppos]],
                                         sem_s.at[jj % NBUF], add=True)
    for jj in range(CPT_ALL - NBUF, CPT_ALL):
      sd[jj % NBUF].wait()
    plsc.subcore_barrier()

    for z in range(4):
      pltpu.async_copy(acc.at[iotav.at[5 + z]], rowbuf.at[0],
                       sem_s.at[0]).wait()
      pltpu.sync_copy(rowbuf.at[0],
                      out_hbm.at[pl.ds(base + sid * WPT + z * CH, CH)])
  return body


def _prop_call(x, src2d, dst2d):
  return pl.kernel(
      _make_prop_body(True),
      out_type=jax.ShapeDtypeStruct((NUM_NODES, CH), jnp.float32),
      mesh=_scmesh(),
      scratch_types=[
          pltpu.VMEM((2, GRP, CH), jnp.int32),
          pltpu.VMEM((2, GRP, CH), jnp.int32),
          pltpu.VMEM((NBUF, CH, CH), jnp.float32),
          pltpu.VMEM((9, CH), jnp.int32),
          pltpu.VMEM_SHARED((ACC_ROWS, CH), jnp.float32),
          pltpu.SemaphoreType.DMA((NBUF,)),
          pltpu.SemaphoreType.DMA((NBUF,)),
          pltpu.SemaphoreType.DMA((2,)),
      ],
  )(x, src2d, dst2d)


def _deg_call(dst2d):
  return pl.kernel(
      _make_prop_body(False),
      out_type=jax.ShapeDtypeStruct((NUM_NODES, CH), jnp.float32),
      mesh=_scmesh(),
      scratch_types=[
          pltpu.VMEM((2, GRP, CH), jnp.int32),
          pltpu.VMEM((1, CH, CH), jnp.float32),
          pltpu.VMEM((9, CH), jnp.int32),
          pltpu.VMEM_SHARED((ACC_ROWS, CH), jnp.float32),
          pltpu.SemaphoreType.DMA((NBUF,)),
          pltpu.SemaphoreType.DMA((2,)),
      ],
  )(dst2d)


def _b_body(x0_ref, degf_ref, w1_ref, th0, th1, dinv_ref):
  deg = degf_ref[:, 0] + 1.0
  dinv = lax.rsqrt(deg)
  t = _dot_tb(x0_ref[...], w1_ref[...])
  ts = t * dinv[:, None]
  dinv_ref[...] = dinv[:, None]
  th0[...] = ts[:, :128]
  th1[...] = ts[:, 128:]


def _b_call(x0, degf, w1):
  rows = N_C
  full = lambda shape: pl.BlockSpec(shape, lambda i: tuple(0 for _ in shape))
  return pl.pallas_call(
      _b_body,
      grid=(NUM_NODES // rows,),
      in_specs=[
          pl.BlockSpec((rows, IN_CH), lambda i: (i, 0)),
          pl.BlockSpec((rows, CH), lambda i: (i, 0)),
          full((IN_CH, IN_CH)),
      ],
      out_specs=[pl.BlockSpec((rows, 128), lambda i: (i, 0))] * 2 +
                [pl.BlockSpec((rows, 1), lambda i: (i, 0))],
      out_shape=[jax.ShapeDtypeStruct((NUM_NODES, 128), jnp.float32)] * 2 +
                [jax.ShapeDtypeStruct((NUM_NODES, 1), jnp.float32)],
  )(x0, degf, w1)


def _c_body(s0, s1, t0, t1, dinv_ref, b1_ref, w2_ref, us_ref):
  dinv = dinv_ref[...]
  h = jnp.concatenate(
      [(s0[...] + t0[...]) * dinv, (s1[...] + t1[...]) * dinv], axis=1)
  h = jnp.maximum(h + b1_ref[...][None, :], 0.0)
  u = _dot_t(h, w2_ref[...])
  us_ref[...] = u * dinv


def _c_call(s_halves, t_halves, dinv, b1, w2p):
  rows = N_C
  full = lambda shape: pl.BlockSpec(shape, lambda i: tuple(0 for _ in shape))
  return pl.pallas_call(
      _c_body,
      grid=(NUM_NODES // rows,),
      in_specs=[pl.BlockSpec((rows, 128), lambda i: (i, 0))] * 4 +
               [pl.BlockSpec((rows, 1), lambda i: (i, 0)),
                full((IN_CH,)), full((CH, IN_CH))],
      out_specs=pl.BlockSpec((rows, CH), lambda i: (i, 0)),
      out_shape=jax.ShapeDtypeStruct((NUM_NODES, CH), jnp.float32),
  )(*s_halves, *t_halves, dinv, b1, w2p)


def _e_body(s2_ref, us_ref, dinv_ref, b2_ref, out_ref):
  res = (s2_ref[...] + us_ref[...]) * dinv_ref[...] + b2_ref[...][None, :]
  out_ref[...] = res[:, :N_CLS]


def _e_call(s2, us, dinv, b2p):
  rows = N_C
  full = lambda shape: pl.BlockSpec(shape, lambda i: tuple(0 for _ in shape))
  return pl.pallas_call(
      _e_body,
      grid=(NUM_NODES // rows,),
      in_specs=[
          pl.BlockSpec((rows, CH), lambda i: (i, 0)),
          pl.BlockSpec((rows, CH), lambda i: (i, 0)),
          pl.BlockSpec((rows, 1), lambda i: (i, 0)),
          full((CH,)),
      ],
      out_specs=pl.BlockSpec((rows, N_CLS), lambda i: (i, 0)),
      out_shape=jax.ShapeDtypeStruct((NUM_NODES, N_CLS), jnp.float32),
  )(s2, us, dinv, b2p)


def kernel(q, k, v, node_features, edge_index, Wq, bq, Wk, bk, Wv, bv, Wo, bo,
           ln_g, ln_b, fc_W, fc_b, g1_W, g1_b, g2_W, g2_b):
  src2d = edge_index[0].astype(jnp.int32).reshape(NCHUNKS, CH)
  dst2d = edge_index[1].astype(jnp.int32).reshape(NCHUNKS, CH)

  degf = _deg_call(dst2d)
  x0 = _attn_call(q, k, v, node_features, Wq, bq, Wk, bk, Wv, bv, Wo, bo,
                  ln_g, ln_b, fc_W, fc_b)

  th0, th1, dinv = _b_call(x0, degf, g1_W)
  s0 = _prop_call(th0, src2d, dst2d)
  s1 = _prop_call(th1, src2d, dst2d)

  w2p = jnp.pad(g2_W, ((0, CH - N_CLS), (0, 0)))
  b2p = jnp.pad(g2_b, (0, CH - N_CLS))
  usp = _c_call((s0, s1), (th0, th1), dinv, g1_b, w2p)
  s2 = _prop_call(usp, src2d, dst2d)
  out = _e_call(s2, usp, dinv, b2p)
  return out.reshape(BS, N_C, N_CLS)


# speedup vs baseline: 15.6749x; 1.0003x over previous
import jax
import jax.numpy as jnp
from jax import lax
from jax.experimental import pallas as pl
from jax.experimental.pallas import tpu as pltpu
from jax.experimental.pallas import tpu_sc as plsc

D_MODEL = 512
N_C = 1024
N_KNOW = 512
BS = 16
NF_DIM = 128
IN_CH = 256
N_CLS = 5
N_EDGES = 262144
NUM_NODES = BS * N_C

NCORE = 2
NSUB = 16
LANES = 16
CH = 128
NCHUNKS = N_EDGES // CH
CPT_ALL = NCHUNKS // NSUB
CPT_HALF = NCHUNKS // (NSUB * NCORE)
NBUF = 3
HALF_NODES = NUM_NODES // NCORE
ACC_ROWS = HALF_NODES + 2 * CH


def _dot_t(a, b):
  return lax.dot_general(a, b, (((1,), (1,)), ((), ())),
                         preferred_element_type=jnp.float32)


def _dot_tb(a, b):
  return lax.dot_general(a.astype(jnp.bfloat16), b.astype(jnp.bfloat16),
                         (((1,), (1,)), ((), ())),
                         preferred_element_type=jnp.float32)


def _scmesh():
  return plsc.VectorSubcoreMesh(core_axis_name="c", subcore_axis_name="s",
                                num_cores=NCORE, num_subcores=NSUB)


def _attn_body(q_ref, k_ref, v_ref, nf_ref, wq_ref, bq_ref, wk_ref, bk_ref,
               wv_ref, bv_ref, wo_ref, bo_ref, lng_ref, lnb_ref, fcw_ref,
               fcb_ref, x0_ref):
  q = q_ref[0]
  k = k_ref[0]
  v = v_ref[0]
  qh = _dot_t(q, wq_ref[...]) + bq_ref[...][None, :]
  kh = _dot_t(k, wk_ref[...]) + bk_ref[...][None, :]
  vh = _dot_t(v, wv_ref[...]) + bv_ref[...][None, :]
  scores = _dot_t(qh, kh) * (1.0 / (D_MODEL ** 0.5))
  m = jnp.max(scores, axis=-1, keepdims=True)
  e = jnp.exp(scores - m)
  attn = e / jnp.sum(e, axis=-1, keepdims=True)
  ctx = jnp.dot(attn, vh, preferred_element_type=jnp.float32)
  out = _dot_t(ctx, wo_ref[...]) + bo_ref[...][None, :]
  qq = q + out
  mu = jnp.mean(qq, axis=-1, keepdims=True)
  var = jnp.mean((qq - mu) ** 2, axis=-1, keepdims=True)
  xln = (qq - mu) / jnp.sqrt(var + 1e-5) * lng_ref[...][None, :] + \
      lnb_ref[...][None, :]
  xcat = jnp.concatenate([xln, nf_ref[...]], axis=1)
  x0_ref[...] = _dot_t(xcat, fcw_ref[...]) + fcb_ref[...][None, :]


def _attn_call(q, k, v, nf, wq, bq, wk, bk, wv, bv, wo, bo, lng, lnb, fcw,
               fcb):
  full = lambda shape: pl.BlockSpec(shape, lambda i: tuple(0 for _ in shape))
  return pl.pallas_call(
      _attn_body,
      grid=(BS,),
      in_specs=[
          pl.BlockSpec((1, N_C, D_MODEL), lambda i: (i, 0, 0)),
          pl.BlockSpec((1, N_KNOW, D_MODEL), lambda i: (i, 0, 0)),
          pl.BlockSpec((1, N_KNOW, D_MODEL), lambda i: (i, 0, 0)),
          full((N_C, NF_DIM)),
          full((D_MODEL, D_MODEL)), full((D_MODEL,)),
          full((D_MODEL, D_MODEL)), full((D_MODEL,)),
          full((D_MODEL, D_MODEL)), full((D_MODEL,)),
          full((D_MODEL, D_MODEL)), full((D_MODEL,)),
          full((D_MODEL,)), full((D_MODEL,)),
          full((IN_CH, D_MODEL + NF_DIM)), full((IN_CH,)),
      ],
      out_specs=pl.BlockSpec((N_C, IN_CH), lambda i: (i, 0)),
      out_shape=jax.ShapeDtypeStruct((NUM_NODES, IN_CH), jnp.float32),
  )(q, k, v, nf, wq, bq, wk, bk, wv, bv, wo, bo, lng, lnb, fcw, fcb)


GRP = 16
NGRP = CPT_ALL // GRP
ZPT = ACC_ROWS // NSUB
WPT = HALF_NODES // NSUB


def _make_prop_body(with_gather):
  def body(*args):
    if with_gather:
      (x_hbm, src_hbm, dst_hbm, out_hbm, srcv, dstloc, rowbuf, iotav, acc,
       sem_g, sem_s, sem_i) = args
    else:
      (dst_hbm, out_hbm, dstloc, rowbuf, iotav, acc,
       sem_s, sem_i) = args
    cid = lax.axis_index("c")
    sid = lax.axis_index("s")
    base = cid * HALF_NODES
    iota = lax.iota(jnp.int32, LANES)
    gbase = sid * CPT_ALL

    def load_group(g, bank, sem):
      ds = pl.ds(gbase + g * GRP, GRP)
      descs = [pltpu.async_copy(dst_hbm.at[ds], dstloc.at[bank], sem)]
      if with_gather:
        descs.append(pltpu.async_copy(src_hbm.at[ds], srcv.at[bank], sem))
      return descs

    def remap_bank(bank):
      def remap(r, carry):
        for vv in range(CH // LANES):
          d = dstloc[bank, r, pl.ds(vv * LANES, LANES)]
          local = d - base
          ok = (local >= 0) & (local < HALF_NODES)
          dummy = HALF_NODES + ((iota + r + vv) & (2 * CH - 1))
          dstloc[bank, r, pl.ds(vv * LANES, LANES)] = jnp.where(
              ok, local, dummy)
        return carry
      lax.fori_loop(0, GRP, remap, 0)

    def ifill(z, carry):
      for vv in range(CH // LANES):
        sl = pl.ds(vv * LANES, LANES)
        lane = vv * LANES + iota
        iotav[z, sl] = sid * ZPT + z * CH + lane
        iotav[5 + z, sl] = sid * WPT + z * CH + lane
      return carry
    lax.fori_loop(0, 4, ifill, 0)
    for vv in range(CH // LANES):
      sl = pl.ds(vv * LANES, LANES)
      lane = vv * LANES + iota
      iotav[4, sl] = jnp.where(4 * CH + lane < ZPT,
                               sid * ZPT + 4 * CH + lane,
                               sid * ZPT + (lane & 7))

    def zfill(i, carry):
      r = i // 8
      c = (i % 8) * LANES
      rowbuf[0, r, pl.ds(c, LANES)] = jnp.zeros((LANES,), jnp.float32)
      return carry
    lax.fori_loop(0, CH * 8, zfill, 0)
    for z in range(5):
      pltpu.async_copy(rowbuf.at[0], acc.at[iotav.at[z]], sem_s.at[0]).wait()
    if not with_gather:
      def ofill(i, carry):
        r = i // 8
        c = (i % 8) * LANES
        rowbuf[0, r, pl.ds(c, LANES)] = jnp.ones((LANES,), jnp.float32)
        return carry
      lax.fori_loop(0, CH * 8, ofill, 0)
    plsc.subcore_barrier()

    for d in load_group(0, 0, sem_i.at[0]):
      d.wait()
    remap_bank(0)

    gd = [None] * NBUF
    sd = [None] * NBUF
    idx_pending = []
    for j in range(CPT_ALL + 2):
      if j < CPT_ALL:
        g, pos = j // GRP, j % GRP
        bank = g % 2
        if pos == 0 and j > 0:
          for d in idx_pending:
            d.wait()
          idx_pending = []
          remap_bank(bank)
        if pos == 3 and g + 1 < NGRP:
          idx_pending = load_group(g + 1, (g + 1) % 2, sem_i.at[(g + 1) % 2])
        if with_gather:
          b = j % NBUF
          if j >= NBUF:
            sd[b].wait()
          gd[b] = pltpu.async_copy(x_hbm.at[srcv.at[bank, pos]],
                                   rowbuf.at[b], sem_g.at[b])
        else:
          b = 0
          if j >= NBUF:
            sd[j % NBUF].wait()
      if j >= 2:
        jj = j - 2
        gg, ppos = jj // GRP, jj % GRP
        bb = (jj % NBUF) if with_gather else 0
        if with_gather:
          gd[bb].wait()
        sd[jj % NBUF] = pltpu.async_copy(rowbuf.at[bb],
                                         acc.at[dstloc.at[gg % 2, ppos]],
                                         sem_s.at[jj % NBUF], add=True)
    for jj in range(CPT_ALL - NBUF, CPT_ALL):
      sd[jj % NBUF].wait()
    plsc.subcore_barrier()

    for z in range(4):
      pltpu.async_copy(acc.at[iotav.at[5 + z]], rowbuf.at[0],
                       sem_s.at[0]).wait()
      pltpu.sync_copy(rowbuf.at[0],
                      out_hbm.at[pl.ds(base + sid * WPT + z * CH, CH)])
  return body


def _prop_call(x, src2d, dst2d):
  return pl.kernel(
      _make_prop_body(True),
      out_type=jax.ShapeDtypeStruct((NUM_NODES, CH), jnp.float32),
      mesh=_scmesh(),
      scratch_types=[
          pltpu.VMEM((2, GRP, CH), jnp.int32),
          pltpu.VMEM((2, GRP, CH), jnp.int32),
          pltpu.VMEM((NBUF, CH, CH), jnp.float32),
          pltpu.VMEM((9, CH), jnp.int32),
          pltpu.VMEM_SHARED((ACC_ROWS, CH), jnp.float32),
          pltpu.SemaphoreType.DMA((NBUF,)),
          pltpu.SemaphoreType.DMA((NBUF,)),
          pltpu.SemaphoreType.DMA((2,)),
      ],
  )(x, src2d, dst2d)


def _deg_call(dst2d):
  return pl.kernel(
      _make_prop_body(False),
      out_type=jax.ShapeDtypeStruct((NUM_NODES, CH), jnp.float32),
      mesh=_scmesh(),
      scratch_types=[
          pltpu.VMEM((2, GRP, CH), jnp.int32),
          pltpu.VMEM((1, CH, CH), jnp.float32),
          pltpu.VMEM((9, CH), jnp.int32),
          pltpu.VMEM_SHARED((ACC_ROWS, CH), jnp.float32),
          pltpu.SemaphoreType.DMA((NBUF,)),
          pltpu.SemaphoreType.DMA((2,)),
      ],
  )(dst2d)


def _b_body(x0_ref, degf_ref, w1_ref, th0, th1, dinv_ref):
  deg = degf_ref[:, 0] + 1.0
  dinv = lax.rsqrt(deg)
  t = _dot_t(x0_ref[...], w1_ref[...])
  ts = t * dinv[:, None]
  dinv_ref[...] = dinv[:, None]
  th0[...] = ts[:, :128]
  th1[...] = ts[:, 128:]


def _b_call(x0, degf, w1):
  rows = N_C
  full = lambda shape: pl.BlockSpec(shape, lambda i: tuple(0 for _ in shape))
  return pl.pallas_call(
      _b_body,
      grid=(NUM_NODES // rows,),
      in_specs=[
          pl.BlockSpec((rows, IN_CH), lambda i: (i, 0)),
          pl.BlockSpec((rows, CH), lambda i: (i, 0)),
          full((IN_CH, IN_CH)),
      ],
      out_specs=[pl.BlockSpec((rows, 128), lambda i: (i, 0))] * 2 +
                [pl.BlockSpec((rows, 1), lambda i: (i, 0))],
      out_shape=[jax.ShapeDtypeStruct((NUM_NODES, 128), jnp.float32)] * 2 +
                [jax.ShapeDtypeStruct((NUM_NODES, 1), jnp.float32)],
  )(x0, degf, w1)


def _c_body(s0, s1, t0, t1, dinv_ref, b1_ref, w2_ref, us_ref):
  dinv = dinv_ref[...]
  h = jnp.concatenate(
      [(s0[...] + t0[...]) * dinv, (s1[...] + t1[...]) * dinv], axis=1)
  h = jnp.maximum(h + b1_ref[...][None, :], 0.0)
  u = _dot_t(h, w2_ref[...])
  us_ref[...] = u * dinv


def _c_call(s_halves, t_halves, dinv, b1, w2p):
  rows = N_C
  full = lambda shape: pl.BlockSpec(shape, lambda i: tuple(0 for _ in shape))
  return pl.pallas_call(
      _c_body,
      grid=(NUM_NODES // rows,),
      in_specs=[pl.BlockSpec((rows, 128), lambda i: (i, 0))] * 4 +
               [pl.BlockSpec((rows, 1), lambda i: (i, 0)),
                full((IN_CH,)), full((CH, IN_CH))],
      out_specs=pl.BlockSpec((rows, CH), lambda i: (i, 0)),
      out_shape=jax.ShapeDtypeStruct((NUM_NODES, CH), jnp.float32),
  )(*s_halves, *t_halves, dinv, b1, w2p)


def _e_body(s2_ref, us_ref, dinv_ref, b2_ref, out_ref):
  res = (s2_ref[...] + us_ref[...]) * dinv_ref[...] + b2_ref[...][None, :]
  out_ref[...] = res[:, :N_CLS]


def _e_call(s2, us, dinv, b2p):
  rows = N_C
  full = lambda shape: pl.BlockSpec(shape, lambda i: tuple(0 for _ in shape))
  return pl.pallas_call(
      _e_body,
      grid=(NUM_NODES // rows,),
      in_specs=[
          pl.BlockSpec((rows, CH), lambda i: (i, 0)),
          pl.BlockSpec((rows, CH), lambda i: (i, 0)),
          pl.BlockSpec((rows, 1), lambda i: (i, 0)),
          full((CH,)),
      ],
      out_specs=pl.BlockSpec((rows, N_CLS), lambda i: (i, 0)),
      out_shape=jax.ShapeDtypeStruct((NUM_NODES, N_CLS), jnp.float32),
  )(s2, us, dinv, b2p)


def kernel(q, k, v, node_features, edge_index, Wq, bq, Wk, bk, Wv, bv, Wo, bo,
           ln_g, ln_b, fc_W, fc_b, g1_W, g1_b, g2_W, g2_b):
  src2d = edge_index[0].astype(jnp.int32).reshape(NCHUNKS, CH)
  dst2d = edge_index[1].astype(jnp.int32).reshape(NCHUNKS, CH)

  degf = _deg_call(dst2d)
  x0 = _attn_call(q, k, v, node_features, Wq, bq, Wk, bk, Wv, bv, Wo, bo,
                  ln_g, ln_b, fc_W, fc_b)

  th0, th1, dinv = _b_call(x0, degf, g1_W)
  s0 = _prop_call(th0, src2d, dst2d)
  s1 = _prop_call(th1, src2d, dst2d)

  w2p = jnp.pad(g2_W, ((0, CH - N_CLS), (0, 0)))
  b2p = jnp.pad(g2_b, (0, CH - N_CLS))
  usp = _c_call((s0, s1), (th0, th1), dinv, g1_b, w2p)
  s2 = _prop_call(usp, src2d, dst2d)
  out = _e_call(s2, usp, dinv, b2p)
  return out.reshape(BS, N_C, N_CLS)
